# baseline (speedup 1.0000x reference)
import jax
import jax.numpy as jnp
from jax import lax
from jax.experimental import pallas as pl
from jax.experimental.pallas import tpu as pltpu
from jax.experimental.pallas import tpu_sc as plsc

N_POINTS = 819200
N_GRID = 2048
N_BASIS = 64
LANES = 16
CHUNK = 512

NUM_CORES = 2
NUM_SUBCORES = 16


def _prep_body(i_ref, td_ref):
    g = i_ref[...]
    g_next = jnp.concatenate([g[1:, :], g[N_GRID - 1:, :]], axis=0)
    td_ref[...] = jnp.concatenate([g, g_next - g], axis=1)


def _prep_table(I_grid):
    return pl.pallas_call(
        _prep_body,
        out_shape=jax.ShapeDtypeStruct((N_GRID, 2 * N_BASIS), jnp.float32),
    )(I_grid)


def _make_sc_interp(n_points, n_grid, n_basis, chunk):
    nw = NUM_CORES * NUM_SUBCORES
    pts_per_w = n_points // nw
    n_iters = pts_per_w // chunk
    ns = NUM_SUBCORES

    def body(t_hbm, td_hbm, out_hbm, t_v, idx_v, w_v, g_v, o_v, sem):
        wid = lax.axis_index("c") * ns + lax.axis_index("s")
        base_w = wid * pts_per_w

        def chunk_body(it, carry):
            base = base_w + it * chunk
            pltpu.sync_copy(t_hbm.at[pl.ds(base, chunk)], t_v)

            def idx_body(j, c):
                sl = pl.ds(j * LANES, LANES)
                tt = t_v[sl]
                u = jnp.clip(tt * float(n_grid - 1), 0.0, float(n_grid - 1))
                i0 = u.astype(jnp.int32)
                idx_v[sl] = i0
                w_v[sl] = u - i0.astype(jnp.float32)
                return c

            lax.fori_loop(0, chunk // LANES, idx_body, 0, unroll=4)

            pltpu.async_copy(td_hbm.at[idx_v], g_v, sem).wait()

            def pt16_body(j, c):
                pb = j * LANES
                pvec = pb + lax.iota(jnp.int32, LANES)
                wv = w_v[pl.ds(pb, LANES)]
                for f in range(n_basis):
                    fa = jnp.full((LANES,), f, jnp.int32)
                    fd = jnp.full((LANES,), n_basis + f, jnp.int32)
                    a = plsc.load_gather(g_v, [pvec, fa])
                    d = plsc.load_gather(g_v, [pvec, fd])
                    o_v[f, pl.ds(pb, LANES)] = a + wv * d
                return c

            lax.fori_loop(0, chunk // LANES, pt16_body, 0)

            pltpu.sync_copy(o_v, out_hbm.at[:, pl.ds(base, chunk)])
            return carry

        lax.fori_loop(0, n_iters, chunk_body, 0)

    return pl.kernel(
        body,
        out_type=jax.ShapeDtypeStruct((n_basis, n_points), jnp.float32),
        mesh=plsc.VectorSubcoreMesh(core_axis_name="c", subcore_axis_name="s",
                                    num_cores=NUM_CORES,
                                    num_subcores=NUM_SUBCORES),
        scratch_types=[
            pltpu.VMEM((chunk,), jnp.float32),
            pltpu.VMEM((chunk,), jnp.int32),
            pltpu.VMEM((chunk + LANES,), jnp.float32),
            pltpu.VMEM((chunk, 2 * n_basis), jnp.float32),
            pltpu.VMEM((n_basis, chunk), jnp.float32),
            pltpu.SemaphoreType.DMA,
        ],
        compiler_params=pltpu.CompilerParams(use_tc_tiling_on_sc=False,
                                             needs_layout_passes=False),
    )


@jax.jit
def kernel(t, I_grid):
    td = _prep_table(I_grid)
    run = _make_sc_interp(N_POINTS, N_GRID, N_BASIS, CHUNK)
    out_t = run(t.reshape(-1), td)
    return out_t.T


# speedup vs baseline: 7.2139x; 7.2139x over previous
import jax
import jax.numpy as jnp
from jax import lax
from jax.experimental import pallas as pl
from jax.experimental.pallas import tpu as pltpu
from jax.experimental.pallas import tpu_sc as plsc

N_POINTS = 819200
N_GRID = 2048
N_BASIS = 64
LANES = 16
CHUNK = 256

NUM_CORES = 2
NUM_SUBCORES = 16


def _prep_body(i_ref, td_ref):
    g = i_ref[...]
    g_next = jnp.concatenate([g[1:, :], g[N_GRID - 1:, :]], axis=0)
    td_ref[...] = jnp.concatenate([g, g_next - g], axis=1)


def _prep_table(I_grid):
    return pl.pallas_call(
        _prep_body,
        out_shape=jax.ShapeDtypeStruct((N_GRID, 2 * N_BASIS), jnp.float32),
    )(I_grid)


def _make_sc_interp(n_points, n_grid, n_basis, chunk):
    nw = NUM_CORES * NUM_SUBCORES
    pts_per_w = n_points // nw
    n_iters = pts_per_w // chunk
    assert n_iters % 2 == 0
    ns = NUM_SUBCORES

    def body(t_hbm, td_hbm, out_hbm,
             t_v0, t_v1, idx_v0, idx_v1, w_v0, w_v1, g_v0, g_v1, o_v0, o_v1,
             tsem0, tsem1, gsem0, gsem1, osem0, osem1):
        t_vs = [t_v0, t_v1]
        idx_vs = [idx_v0, idx_v1]
        w_vs = [w_v0, w_v1]
        g_vs = [g_v0, g_v1]
        o_vs = [o_v0, o_v1]
        tsem = [tsem0, tsem1]
        gsem = [gsem0, gsem1]
        osem = [osem0, osem1]

        wid = lax.axis_index("c") * ns + lax.axis_index("s")
        base_w = wid * pts_per_w
        base_w2 = wid * (pts_per_w // 2)

        def t_slice(it):
            return t_hbm.at[pl.ds(base_w + it * chunk, chunk)]

        def out_slice(it):
            return out_hbm.at[pl.ds(base_w2 + it * (chunk // 2), chunk // 2)]

        def idx_compute(s):
            t_v, idx_v, w_v = t_vs[s], idx_vs[s], w_vs[s]

            def idx_body(j, c):
                sl = pl.ds(j * LANES, LANES)
                tt = t_v[sl]
                u = jnp.clip(tt * float(n_grid - 1), 0.0, float(n_grid - 1))
                i0 = u.astype(jnp.int32)
                idx_v[sl] = i0
                w_v[sl] = u - i0.astype(jnp.float32)
                return c

            lax.fori_loop(0, chunk // LANES, idx_body, 0, unroll=4)

        def interp(s):
            w_v, g_v, o_v = w_vs[s], g_vs[s], o_vs[s]

            def pt_body(p, c):
                w = w_v[pl.ds(p, LANES)][0]
                r = p >> 1
                cb = (p & 1) * n_basis
                for f in range(n_basis // LANES):
                    a = g_v[p, pl.ds(f * LANES, LANES)]
                    d = g_v[p, pl.ds(n_basis + f * LANES, LANES)]
                    o_v[r, pl.ds(cb + f * LANES, LANES)] = a + w * d
                return c

            lax.fori_loop(0, chunk, pt_body, 0, unroll=8)

        pltpu.async_copy(t_slice(0), t_vs[0], tsem[0])
        pltpu.async_copy(t_slice(1), t_vs[1], tsem[1])
        pltpu.make_async_copy(t_slice(0), t_vs[0], tsem[0]).wait()
        idx_compute(0)
        pltpu.async_copy(td_hbm.at[idx_vs[0]], g_vs[0], gsem[0])

        def pair_body(q, carry):
            for b in (0, 1):
                it = 2 * q + b
                s, nxt = b, 1 - b

                @pl.when(it + 1 < n_iters)
                def _():
                    pltpu.make_async_copy(
                        t_slice(it + 1), t_vs[nxt], tsem[nxt]).wait()
                    idx_compute(nxt)
                    pltpu.async_copy(
                        td_hbm.at[idx_vs[nxt]], g_vs[nxt], gsem[nxt])

                @pl.when(it + 2 < n_iters)
                def _():
                    pltpu.async_copy(t_slice(it + 2), t_vs[s], tsem[s])

                pltpu.make_async_copy(
                    td_hbm.at[idx_vs[s]], g_vs[s], gsem[s]).wait()

                @pl.when(it >= 2)
                def _():
                    pltpu.make_async_copy(
                        o_vs[s], out_slice(it), osem[s]).wait()

                interp(s)
                pltpu.async_copy(o_vs[s], out_slice(it), osem[s])
            return carry

        lax.fori_loop(0, n_iters // 2, pair_body, 0)

        pltpu.make_async_copy(o_vs[0], out_slice(n_iters - 2), osem[0]).wait()
        pltpu.make_async_copy(o_vs[1], out_slice(n_iters - 1), osem[1]).wait()

    return pl.kernel(
        body,
        out_type=jax.ShapeDtypeStruct((n_points // 2, 2 * n_basis), jnp.float32),
        mesh=plsc.VectorSubcoreMesh(core_axis_name="c", subcore_axis_name="s",
                                    num_cores=NUM_CORES,
                                    num_subcores=NUM_SUBCORES),
        scratch_types=[
            pltpu.VMEM((chunk,), jnp.float32),
            pltpu.VMEM((chunk,), jnp.float32),
            pltpu.VMEM((chunk,), jnp.int32),
            pltpu.VMEM((chunk,), jnp.int32),
            pltpu.VMEM((chunk + LANES,), jnp.float32),
            pltpu.VMEM((chunk + LANES,), jnp.float32),
            pltpu.VMEM((chunk, 2 * n_basis), jnp.float32),
            pltpu.VMEM((chunk, 2 * n_basis), jnp.float32),
            pltpu.VMEM((chunk // 2, 2 * n_basis), jnp.float32),
            pltpu.VMEM((chunk // 2, 2 * n_basis), jnp.float32),
            pltpu.SemaphoreType.DMA,
            pltpu.SemaphoreType.DMA,
            pltpu.SemaphoreType.DMA,
            pltpu.SemaphoreType.DMA,
            pltpu.SemaphoreType.DMA,
            pltpu.SemaphoreType.DMA,
        ],
        compiler_params=pltpu.CompilerParams(use_tc_tiling_on_sc=False),
    )


@jax.jit
def kernel(t, I_grid):
    td = _prep_table(I_grid)
    run = _make_sc_interp(N_POINTS, N_GRID, N_BASIS, CHUNK)
    out2 = run(t.reshape(-1), td)
    return out2.reshape(N_POINTS, N_BASIS)


# speedup vs baseline: 14.7626x; 2.0464x over previous
import jax
import jax.numpy as jnp
from jax import lax
from jax.experimental import pallas as pl
from jax.experimental.pallas import tpu as pltpu
from jax.experimental.pallas import tpu_sc as plsc

N_POINTS = 819200
N_GRID = 2048
N_BASIS = 64
LANES = 16
CHUNK = 256

NUM_CORES = 2
NUM_SUBCORES = 16


def _prep_body(i_ref, td_ref):
    g = i_ref[...]
    g_next = jnp.concatenate([g[1:, :], g[N_GRID - 1:, :]], axis=0)
    td_ref[...] = jnp.concatenate([g, g_next - g], axis=1)


def _prep_table(I_grid):
    return pl.pallas_call(
        _prep_body,
        out_shape=jax.ShapeDtypeStruct((N_GRID, 2 * N_BASIS), jnp.float32),
    )(I_grid)


def _make_sc_interp(n_points, n_grid, n_basis, chunk):
    nw = NUM_CORES * NUM_SUBCORES
    pts_per_w = n_points // nw
    n_iters = pts_per_w // chunk
    assert n_iters % 2 == 0
    ns = NUM_SUBCORES

    def body(t_hbm, td_hbm, out_hbm,
             t_v0, t_v1, idx_v0, idx_v1, w_v0, w_v1, g_v0, g_v1,
             tsem0, tsem1, gsem0, gsem1, osem0, osem1):
        t_vs = [t_v0, t_v1]
        idx_vs = [idx_v0, idx_v1]
        w_vs = [w_v0, w_v1]
        g_vs = [g_v0, g_v1]
        tsem = [tsem0, tsem1]
        gsem = [gsem0, gsem1]
        osem = [osem0, osem1]

        wid = lax.axis_index("c") * ns + lax.axis_index("s")
        base_w = wid * pts_per_w

        def t_slice(it):
            return t_hbm.at[pl.ds(base_w + it * chunk, chunk)]

        def out_slice(it):
            return out_hbm.at[pl.ds(base_w + it * chunk, chunk),
                              pl.ds(0, n_basis)]

        def o_src(s):
            return g_vs[s].at[:, pl.ds(0, n_basis)]

        def idx_compute(s):
            t_v, idx_v, w_v = t_vs[s], idx_vs[s], w_vs[s]

            def idx_body(j, c):
                sl = pl.ds(j * LANES, LANES)
                tt = t_v[sl]
                u = jnp.clip(tt * float(n_grid - 1), 0.0, float(n_grid - 1))
                i0 = u.astype(jnp.int32)
                idx_v[sl] = i0
                w_v[sl] = u - i0.astype(jnp.float32)
                return c

            lax.fori_loop(0, chunk // LANES, idx_body, 0, unroll=4)

        def interp(s):
            w_v, g_v = w_vs[s], g_vs[s]

            def pt_body(p, c):
                w = w_v[pl.ds(p, LANES)][0]
                for f in range(n_basis // LANES):
                    a = g_v[p, pl.ds(f * LANES, LANES)]
                    d = g_v[p, pl.ds(n_basis + f * LANES, LANES)]
                    g_v[p, pl.ds(f * LANES, LANES)] = a + w * d
                return c

            lax.fori_loop(0, chunk, pt_body, 0, unroll=8)

        pltpu.async_copy(t_slice(0), t_vs[0], tsem[0])
        pltpu.async_copy(t_slice(1), t_vs[1], tsem[1])
        pltpu.make_async_copy(t_slice(0), t_vs[0], tsem[0]).wait()
        idx_compute(0)
        pltpu.async_copy(td_hbm.at[idx_vs[0]], g_vs[0], gsem[0])

        def pair_body(q, carry):
            for b in (0, 1):
                it = 2 * q + b
                s, nxt = b, 1 - b

                @pl.when(it + 1 < n_iters)
                def _():
                    pltpu.make_async_copy(
                        t_slice(it + 1), t_vs[nxt], tsem[nxt]).wait()
                    idx_compute(nxt)

                    @pl.when(it >= 1)
                    def _():
                        pltpu.make_async_copy(
                            o_src(nxt), out_slice(it - 1), osem[nxt]).wait()

                    pltpu.async_copy(
                        td_hbm.at[idx_vs[nxt]], g_vs[nxt], gsem[nxt])

                @pl.when(it + 2 < n_iters)
                def _():
                    pltpu.async_copy(t_slice(it + 2), t_vs[s], tsem[s])

                pltpu.make_async_copy(
                    td_hbm.at[idx_vs[s]], g_vs[s], gsem[s]).wait()

                interp(s)
                pltpu.async_copy(o_src(s), out_slice(it), osem[s])
            return carry

        lax.fori_loop(0, n_iters // 2, pair_body, 0)

        pltpu.make_async_copy(o_src(0), out_slice(n_iters - 2), osem[0]).wait()
        pltpu.make_async_copy(o_src(1), out_slice(n_iters - 1), osem[1]).wait()

    return pl.kernel(
        body,
        out_type=jax.ShapeDtypeStruct((n_points, 2 * n_basis), jnp.float32),
        mesh=plsc.VectorSubcoreMesh(core_axis_name="c", subcore_axis_name="s",
                                    num_cores=NUM_CORES,
                                    num_subcores=NUM_SUBCORES),
        scratch_types=[
            pltpu.VMEM((chunk,), jnp.float32),
            pltpu.VMEM((chunk,), jnp.float32),
            pltpu.VMEM((chunk,), jnp.int32),
            pltpu.VMEM((chunk,), jnp.int32),
            pltpu.VMEM((chunk + LANES,), jnp.float32),
            pltpu.VMEM((chunk + LANES,), jnp.float32),
            pltpu.VMEM((chunk, 2 * n_basis), jnp.float32),
            pltpu.VMEM((chunk, 2 * n_basis), jnp.float32),
            pltpu.SemaphoreType.DMA,
            pltpu.SemaphoreType.DMA,
            pltpu.SemaphoreType.DMA,
            pltpu.SemaphoreType.DMA,
            pltpu.SemaphoreType.DMA,
            pltpu.SemaphoreType.DMA,
        ],
        compiler_params=pltpu.CompilerParams(use_tc_tiling_on_sc=False),
    )


@jax.jit
def kernel(t, I_grid):
    td = _prep_table(I_grid)
    run = _make_sc_interp(N_POINTS, N_GRID, N_BASIS, CHUNK)
    out128 = run(t.reshape(-1), td)
    return out128[:, :N_BASIS]


# speedup vs baseline: 14.8052x; 1.0029x over previous
import jax
import jax.numpy as jnp
from jax import lax
from jax.experimental import pallas as pl
from jax.experimental.pallas import tpu as pltpu
from jax.experimental.pallas import tpu_sc as plsc

N_POINTS = 819200
N_GRID = 2048
N_BASIS = 64
LANES = 16
CHUNK = 320

NUM_CORES = 2
NUM_SUBCORES = 16


def _prep_body(i_ref, td_ref):
    g = i_ref[...]
    g_next = jnp.concatenate([g[1:, :], g[N_GRID - 1:, :]], axis=0)
    td_ref[...] = jnp.concatenate([g, g_next - g], axis=1)


def _prep_table(I_grid):
    return pl.pallas_call(
        _prep_body,
        out_shape=jax.ShapeDtypeStruct((N_GRID, 2 * N_BASIS), jnp.float32),
    )(I_grid)


def _make_sc_interp(n_points, n_grid, n_basis, chunk):
    nw = NUM_CORES * NUM_SUBCORES
    pts_per_w = n_points // nw
    n_iters = pts_per_w // chunk
    assert n_iters % 2 == 0
    ns = NUM_SUBCORES

    def body(t_hbm, td_hbm, out_hbm,
             t_v0, t_v1, idx_v0, idx_v1, w_v0, w_v1, g_v0, g_v1,
             tsem0, tsem1, gsem0, gsem1, osem0, osem1):
        t_vs = [t_v0, t_v1]
        idx_vs = [idx_v0, idx_v1]
        w_vs = [w_v0, w_v1]
        g_vs = [g_v0, g_v1]
        tsem = [tsem0, tsem1]
        gsem = [gsem0, gsem1]
        osem = [osem0, osem1]

        wid = lax.axis_index("c") * ns + lax.axis_index("s")
        base_w = wid * pts_per_w

        def t_slice(it):
            return t_hbm.at[pl.ds(base_w + it * chunk, chunk)]

        def out_slice(it):
            return out_hbm.at[pl.ds(base_w + it * chunk, chunk),
                              pl.ds(0, n_basis)]

        def o_src(s):
            return g_vs[s].at[:, pl.ds(0, n_basis)]

        def idx_compute(s):
            t_v, idx_v, w_v = t_vs[s], idx_vs[s], w_vs[s]

            def idx_body(j, c):
                sl = pl.ds(j * LANES, LANES)
                tt = t_v[sl]
                u = jnp.clip(tt * float(n_grid - 1), 0.0, float(n_grid - 1))
                i0 = u.astype(jnp.int32)
                idx_v[sl] = i0
                w_v[sl] = u - i0.astype(jnp.float32)
                return c

            lax.fori_loop(0, chunk // LANES, idx_body, 0, unroll=4)

        def interp(s):
            w_v, g_v = w_vs[s], g_vs[s]

            def pt_body(p, c):
                w = w_v[pl.ds(p, LANES)][0]
                for f in range(n_basis // LANES):
                    a = g_v[p, pl.ds(f * LANES, LANES)]
                    d = g_v[p, pl.ds(n_basis + f * LANES, LANES)]
                    g_v[p, pl.ds(f * LANES, LANES)] = a + w * d
                return c

            lax.fori_loop(0, chunk, pt_body, 0, unroll=8)

        pltpu.async_copy(t_slice(0), t_vs[0], tsem[0])
        pltpu.async_copy(t_slice(1), t_vs[1], tsem[1])
        pltpu.make_async_copy(t_slice(0), t_vs[0], tsem[0]).wait()
        idx_compute(0)
        pltpu.async_copy(td_hbm.at[idx_vs[0]], g_vs[0], gsem[0])

        def pair_body(q, carry):
            for b in (0, 1):
                it = 2 * q + b
                s, nxt = b, 1 - b

                @pl.when(it + 1 < n_iters)
                def _():
                    pltpu.make_async_copy(
                        t_slice(it + 1), t_vs[nxt], tsem[nxt]).wait()
                    idx_compute(nxt)

                    @pl.when(it >= 1)
                    def _():
                        pltpu.make_async_copy(
                            o_src(nxt), out_slice(it - 1), osem[nxt]).wait()

                    pltpu.async_copy(
                        td_hbm.at[idx_vs[nxt]], g_vs[nxt], gsem[nxt])

                @pl.when(it + 2 < n_iters)
                def _():
                    pltpu.async_copy(t_slice(it + 2), t_vs[s], tsem[s])

                pltpu.make_async_copy(
                    td_hbm.at[idx_vs[s]], g_vs[s], gsem[s]).wait()

                interp(s)
                pltpu.async_copy(o_src(s), out_slice(it), osem[s])
            return carry

        lax.fori_loop(0, n_iters // 2, pair_body, 0)

        pltpu.make_async_copy(o_src(0), out_slice(n_iters - 2), osem[0]).wait()
        pltpu.make_async_copy(o_src(1), out_slice(n_iters - 1), osem[1]).wait()

    return pl.kernel(
        body,
        out_type=jax.ShapeDtypeStruct((n_points, 2 * n_basis), jnp.float32),
        mesh=plsc.VectorSubcoreMesh(core_axis_name="c", subcore_axis_name="s",
                                    num_cores=NUM_CORES,
                                    num_subcores=NUM_SUBCORES),
        scratch_types=[
            pltpu.VMEM((chunk,), jnp.float32),
            pltpu.VMEM((chunk,), jnp.float32),
            pltpu.VMEM((chunk,), jnp.int32),
            pltpu.VMEM((chunk,), jnp.int32),
            pltpu.VMEM((chunk + LANES,), jnp.float32),
            pltpu.VMEM((chunk + LANES,), jnp.float32),
            pltpu.VMEM((chunk, 2 * n_basis), jnp.float32),
            pltpu.VMEM((chunk, 2 * n_basis), jnp.float32),
            pltpu.SemaphoreType.DMA,
            pltpu.SemaphoreType.DMA,
            pltpu.SemaphoreType.DMA,
            pltpu.SemaphoreType.DMA,
            pltpu.SemaphoreType.DMA,
            pltpu.SemaphoreType.DMA,
        ],
        compiler_params=pltpu.CompilerParams(use_tc_tiling_on_sc=False),
    )


@jax.jit
def kernel(t, I_grid):
    td = _prep_table(I_grid)
    run = _make_sc_interp(N_POINTS, N_GRID, N_BASIS, CHUNK)
    out128 = run(t.reshape(-1), td)
    return out128[:, :N_BASIS]
